# baseline (speedup 1.0000x reference)
import jax
import jax.numpy as jnp
from jax.experimental import pallas as pl
from jax.experimental.pallas import tpu as pltpu

EMBED_DIM = 16


def _body(idx_ref, table_ref, out_ref):
    out_ref[...] = jnp.full((1, EMBED_DIM), idx_ref[0], jnp.float32)


def kernel(client_id, embed_table):
    idx = jnp.asarray(client_id, dtype=jnp.int32).reshape((1,))
    return pl.pallas_call(
        _body,
        in_specs=[
            pl.BlockSpec(memory_space=pltpu.SMEM),
            pl.BlockSpec(memory_space=pl.ANY),
        ],
        out_shape=jax.ShapeDtypeStruct((1, EMBED_DIM), jnp.float32),
    )(idx, embed_table.reshape(16_000_000))


# speedup vs baseline: 223.4145x; 223.4145x over previous
import jax
import jax.numpy as jnp
from jax.experimental import pallas as pl
from jax.experimental.pallas import tpu as pltpu

EMBED_DIM = 16
LANES = 128


def _body(idx_ref, table_ref, out_ref):
    c = idx_ref[0] % LANES
    rolled = pltpu.roll(table_ref[...], -c, 1)
    out_ref[...] = jnp.swapaxes(rolled[:, :1], 0, 1)


def kernel(client_id, embed_table):
    idx = jnp.asarray(client_id, dtype=jnp.int32).reshape((1,))
    grid_spec = pltpu.PrefetchScalarGridSpec(
        num_scalar_prefetch=1,
        grid=(1,),
        in_specs=[
            pl.BlockSpec(
                (EMBED_DIM, LANES),
                lambda i, idx_ref: (0, idx_ref[0] // LANES),
            ),
        ],
        out_specs=pl.BlockSpec((1, EMBED_DIM), lambda i, idx_ref: (0, 0)),
    )
    return pl.pallas_call(
        _body,
        grid_spec=grid_spec,
        out_shape=jax.ShapeDtypeStruct((1, EMBED_DIM), jnp.float32),
    )(idx, embed_table.T)
